# Initial kernel scaffold; baseline (speedup 1.0000x reference)
import jax
import jax.numpy as jnp
from jax.experimental import pallas as pl


def kernel(volume, t_sorted, M, b, src, dst):
    raise NotImplementedError("write your pallas kernel here")


# speedup vs baseline: 1.1176x; 1.1176x over previous
import functools

import jax
import jax.numpy as jnp
from jax import lax
from jax.experimental import pallas as pl
from jax.experimental.pallas import tpu as pltpu
from jax.experimental.pallas import tpu_sc as plsc


def _geom_body(scal_ref, t_ref, sx_ref, sy_ref, sz_ref, ex_ref, ey_ref,
               ez_ref, idx_ref, w_ref, *, n_x, n_y, n_z):
    n_int = t_ref.shape[0]
    t0 = t_ref[...]
    t1 = jnp.concatenate([t0[1:n_int], t0[n_int - 1:n_int]], axis=0)

    sx = sx_ref[...]; sy = sy_ref[...]; sz = sz_ref[...]
    dx = ex_ref[...] - sx
    dy = ey_ref[...] - sy
    dz = ez_ref[...] - sz

    x0 = sx + t0 * dx; x1 = sx + t1 * dx
    y0 = sy + t0 * dy; y1 = sy + t1 * dy
    z0 = sz + t0 * dz; z1 = sz + t1 * dz

    seg_len = jnp.sqrt((x1 - x0) ** 2 + (y1 - y0) ** 2 + (z1 - z0) ** 2)

    mx = 0.5 * (x0 + x1)
    my = 0.5 * (y0 + y1)
    mz = 0.5 * (z0 + z1)
    mxs = mx - scal_ref[9]
    mys = my - scal_ref[10]
    mzs = mz - scal_ref[11]
    i_f = scal_ref[0] * mxs + scal_ref[1] * mys + scal_ref[2] * mzs
    j_f = scal_ref[3] * mxs + scal_ref[4] * mys + scal_ref[5] * mzs
    k_f = scal_ref[6] * mxs + scal_ref[7] * mys + scal_ref[8] * mzs
    i_i = jnp.round(i_f).astype(jnp.int32)
    j_i = jnp.round(j_f).astype(jnp.int32)
    k_i = jnp.round(k_f).astype(jnp.int32)
    oob = ((i_i < 0) | (i_i >= n_x) | (j_i < 0) | (j_i >= n_y)
           | (k_i < 0) | (k_i >= n_z))
    flat = i_i * (n_y * n_z) + j_i * n_z + k_i
    idx_ref[0] = jnp.where(oob, 0, flat)
    w_ref[0] = jnp.where(oob, 0.0, seg_len)


def _geometry(t_sorted, src, dst, m_inv, b, n_x, n_y, n_z, block_rays):
    n_ray, n_int = t_sorted.shape
    nb = n_ray // block_rays
    t_t = t_sorted.T
    sx = src[:, 0].reshape(1, n_ray); sy = src[:, 1].reshape(1, n_ray)
    sz = src[:, 2].reshape(1, n_ray)
    ex = dst[:, 0].reshape(1, n_ray); ey = dst[:, 1].reshape(1, n_ray)
    ez = dst[:, 2].reshape(1, n_ray)
    scal = jnp.concatenate([m_inv.reshape(-1), b]).astype(jnp.float32)

    ray_spec = pl.BlockSpec((1, block_rays), lambda i: (0, i))
    out_spec = pl.BlockSpec((1, n_int, block_rays), lambda i: (i, 0, 0))
    return pl.pallas_call(
        functools.partial(_geom_body, n_x=n_x, n_y=n_y, n_z=n_z),
        grid=(nb,),
        in_specs=[
            pl.BlockSpec(memory_space=pltpu.SMEM),
            pl.BlockSpec((n_int, block_rays), lambda i: (0, i)),
            ray_spec, ray_spec, ray_spec, ray_spec, ray_spec, ray_spec,
        ],
        out_specs=[out_spec, out_spec],
        out_shape=[
            jax.ShapeDtypeStruct((nb, n_int, block_rays), jnp.int32),
            jax.ShapeDtypeStruct((nb, n_int, block_rays), jnp.float32),
        ],
    )(scal, t_t, sx, sy, sz, ex, ey, ez)


def _sc_gather(vol_flat, idx, w, n_ray, n_int, block_rays):
    nb = idx.shape[0]
    blk_elems = n_int * block_rays
    idx = idx.reshape(nb, blk_elems)
    w = w.reshape(nb, blk_elems)
    info = plsc.get_sparse_core_info()
    nc, ns, nl = info.num_cores, info.num_subcores, info.num_lanes
    nw = nc * ns
    chunks_per_worker = nb // nw
    ngroups = block_rays // nl

    mesh = plsc.VectorSubcoreMesh(core_axis_name="c", subcore_axis_name="s")

    @functools.partial(
        pl.kernel,
        out_type=jax.ShapeDtypeStruct((n_ray,), jnp.float32),
        mesh=mesh,
        scratch_types=[
            pltpu.VMEM((blk_elems,), jnp.int32),
            pltpu.VMEM((blk_elems,), jnp.float32),
            pltpu.VMEM((blk_elems,), jnp.float32),
            pltpu.VMEM((block_rays,), jnp.float32),
            pltpu.SemaphoreType.DMA,
        ],
    )
    def run(vol_hbm, idx_hbm, w_hbm, out_hbm, idx_v, w_v, vals_v, out_v, sem):
        wid = lax.axis_index("s") * nc + lax.axis_index("c")

        def chunk_body(c, _):
            blk = wid * chunks_per_worker + c
            pltpu.sync_copy(idx_hbm.at[blk], idx_v)
            pltpu.sync_copy(w_hbm.at[blk], w_v)
            pltpu.async_copy(vol_hbm.at[idx_v], vals_v, sem).wait()

            def seg_body(s, accs):
                base = s * block_rays
                return tuple(
                    accs[g] + vals_v[pl.ds(base + g * nl, nl)]
                    * w_v[pl.ds(base + g * nl, nl)]
                    for g in range(ngroups)
                )

            accs = tuple(jnp.zeros((nl,), jnp.float32) for _ in range(ngroups))
            accs = lax.fori_loop(0, n_int, seg_body, accs)
            for g in range(ngroups):
                out_v[g * nl:(g + 1) * nl] = accs[g]
            pltpu.sync_copy(out_v, out_hbm.at[pl.ds(blk * block_rays,
                                                    block_rays)])
            return 0

        lax.fori_loop(0, chunks_per_worker, chunk_body, 0)

    return run(vol_flat, idx, w)


def kernel(volume, t_sorted, M, b, src, dst):
    n_x, n_y, n_z = volume.shape
    n_ray, n_int = t_sorted.shape
    m_inv = jnp.linalg.inv(M)
    block_rays = 128
    idx, w = _geometry(t_sorted, src, dst, m_inv, b, n_x, n_y, n_z,
                       block_rays)
    return _sc_gather(volume.reshape(-1), idx, w, n_ray, n_int, block_rays)


# speedup vs baseline: 1.2617x; 1.1289x over previous
import functools

import jax
import jax.numpy as jnp
from jax import lax
from jax.experimental import pallas as pl
from jax.experimental.pallas import tpu as pltpu
from jax.experimental.pallas import tpu_sc as plsc


def _geom_body(scal_ref, t_ref, sx_ref, sy_ref, sz_ref, ex_ref, ey_ref,
               ez_ref, idx_ref, w_ref, *, n_x, n_y, n_z):
    n_int = t_ref.shape[1]
    t0 = t_ref[...].T
    t1 = jnp.concatenate([t0[1:n_int], t0[n_int - 1:n_int]], axis=0)

    sx = sx_ref[...]; sy = sy_ref[...]; sz = sz_ref[...]
    dx = ex_ref[...] - sx
    dy = ey_ref[...] - sy
    dz = ez_ref[...] - sz

    x0 = sx + t0 * dx; x1 = sx + t1 * dx
    y0 = sy + t0 * dy; y1 = sy + t1 * dy
    z0 = sz + t0 * dz; z1 = sz + t1 * dz

    seg_len = jnp.sqrt((x1 - x0) ** 2 + (y1 - y0) ** 2 + (z1 - z0) ** 2)

    mx = 0.5 * (x0 + x1)
    my = 0.5 * (y0 + y1)
    mz = 0.5 * (z0 + z1)
    mxs = mx - scal_ref[9]
    mys = my - scal_ref[10]
    mzs = mz - scal_ref[11]
    i_f = scal_ref[0] * mxs + scal_ref[1] * mys + scal_ref[2] * mzs
    j_f = scal_ref[3] * mxs + scal_ref[4] * mys + scal_ref[5] * mzs
    k_f = scal_ref[6] * mxs + scal_ref[7] * mys + scal_ref[8] * mzs
    i_i = jnp.round(i_f).astype(jnp.int32)
    j_i = jnp.round(j_f).astype(jnp.int32)
    k_i = jnp.round(k_f).astype(jnp.int32)
    oob = ((i_i < 0) | (i_i >= n_x) | (j_i < 0) | (j_i >= n_y)
           | (k_i < 0) | (k_i >= n_z))
    flat = i_i * (n_y * n_z) + j_i * n_z + k_i
    idx_ref[...] = jnp.where(oob, 0, flat)
    w_ref[...] = jnp.where(oob, 0.0, seg_len)


def _geometry(t_sorted, src, dst, m_inv, b, n_x, n_y, n_z, block_rays):
    n_ray, n_int = t_sorted.shape
    nb = n_ray // block_rays
    sx = src[:, 0].reshape(1, n_ray); sy = src[:, 1].reshape(1, n_ray)
    sz = src[:, 2].reshape(1, n_ray)
    ex = dst[:, 0].reshape(1, n_ray); ey = dst[:, 1].reshape(1, n_ray)
    ez = dst[:, 2].reshape(1, n_ray)
    scal = jnp.concatenate([m_inv.reshape(-1), b]).astype(jnp.float32)

    ray_spec = pl.BlockSpec((1, block_rays), lambda i: (0, i))
    out_spec = pl.BlockSpec((n_int, block_rays), lambda i: (i, 0))
    return pl.pallas_call(
        functools.partial(_geom_body, n_x=n_x, n_y=n_y, n_z=n_z),
        grid=(nb,),
        in_specs=[
            pl.BlockSpec(memory_space=pltpu.SMEM),
            pl.BlockSpec((block_rays, n_int), lambda i: (i, 0)),
            ray_spec, ray_spec, ray_spec, ray_spec, ray_spec, ray_spec,
        ],
        out_specs=[out_spec, out_spec],
        out_shape=[
            jax.ShapeDtypeStruct((nb * n_int, block_rays), jnp.int32),
            jax.ShapeDtypeStruct((nb * n_int, block_rays), jnp.float32),
        ],
    )(scal, t_sorted, sx, sy, sz, ex, ey, ez)


def _sc_gather(vol_flat, idx, w, n_ray, n_int, block_rays):
    blk_elems = n_int * block_rays
    nb = idx.shape[0] // blk_elems
    info = plsc.get_sparse_core_info()
    nc, ns, nl = info.num_cores, info.num_subcores, info.num_lanes
    nw = nc * ns
    chunks_per_worker = nb // nw
    ngroups = block_rays // nl

    mesh = plsc.VectorSubcoreMesh(core_axis_name="c", subcore_axis_name="s")

    @functools.partial(
        pl.kernel,
        out_type=jax.ShapeDtypeStruct((n_ray,), jnp.float32),
        mesh=mesh,
        scratch_types=[
            pltpu.VMEM((blk_elems,), jnp.int32),
            pltpu.VMEM((blk_elems,), jnp.float32),
            pltpu.VMEM((blk_elems,), jnp.float32),
            pltpu.VMEM((block_rays,), jnp.float32),
            pltpu.SemaphoreType.DMA,
        ],
    )
    def run(vol_hbm, idx_hbm, w_hbm, out_hbm, idx_v, w_v, vals_v, out_v, sem):
        wid = lax.axis_index("s") * nc + lax.axis_index("c")

        def chunk_body(c, _):
            blk = wid * chunks_per_worker + c
            el0 = blk * blk_elems
            pltpu.sync_copy(idx_hbm.at[pl.ds(el0, blk_elems)], idx_v)
            pltpu.sync_copy(w_hbm.at[pl.ds(el0, blk_elems)], w_v)
            pltpu.async_copy(vol_hbm.at[idx_v], vals_v, sem).wait()

            def seg_body(s, accs):
                base = s * block_rays
                return tuple(
                    accs[g] + vals_v[pl.ds(base + g * nl, nl)]
                    * w_v[pl.ds(base + g * nl, nl)]
                    for g in range(ngroups)
                )

            accs = tuple(jnp.zeros((nl,), jnp.float32) for _ in range(ngroups))
            accs = lax.fori_loop(0, n_int, seg_body, accs)
            for g in range(ngroups):
                out_v[g * nl:(g + 1) * nl] = accs[g]
            pltpu.sync_copy(out_v, out_hbm.at[pl.ds(blk * block_rays,
                                                    block_rays)])
            return 0

        lax.fori_loop(0, chunks_per_worker, chunk_body, 0)

    return run(vol_flat, idx, w)


def kernel(volume, t_sorted, M, b, src, dst):
    n_x, n_y, n_z = volume.shape
    n_ray, n_int = t_sorted.shape
    m_inv = jnp.linalg.inv(M)
    block_rays = 128
    idx, w = _geometry(t_sorted, src, dst, m_inv, b, n_x, n_y, n_z,
                       block_rays)
    return _sc_gather(volume.reshape(-1), idx.reshape(-1), w.reshape(-1),
                      n_ray, n_int, block_rays)


# speedup vs baseline: 1.7850x; 1.4147x over previous
import functools

import jax
import jax.numpy as jnp
from jax import lax
from jax.experimental import pallas as pl
from jax.experimental.pallas import tpu as pltpu
from jax.experimental.pallas import tpu_sc as plsc


def _geom_body(scal_ref, t_ref, sx_ref, sy_ref, sz_ref, ex_ref, ey_ref,
               ez_ref, idx_ref, w_ref, *, n_x, n_y, n_z):
    n_int = t_ref.shape[1]
    t0 = t_ref[...].T
    t1 = jnp.concatenate([t0[1:n_int], t0[n_int - 1:n_int]], axis=0)

    sx = sx_ref[...]; sy = sy_ref[...]; sz = sz_ref[...]
    dx = ex_ref[...] - sx
    dy = ey_ref[...] - sy
    dz = ez_ref[...] - sz

    x0 = sx + t0 * dx; x1 = sx + t1 * dx
    y0 = sy + t0 * dy; y1 = sy + t1 * dy
    z0 = sz + t0 * dz; z1 = sz + t1 * dz

    seg_len = jnp.sqrt((x1 - x0) ** 2 + (y1 - y0) ** 2 + (z1 - z0) ** 2)

    mx = 0.5 * (x0 + x1)
    my = 0.5 * (y0 + y1)
    mz = 0.5 * (z0 + z1)
    mxs = mx - scal_ref[9]
    mys = my - scal_ref[10]
    mzs = mz - scal_ref[11]
    i_f = scal_ref[0] * mxs + scal_ref[1] * mys + scal_ref[2] * mzs
    j_f = scal_ref[3] * mxs + scal_ref[4] * mys + scal_ref[5] * mzs
    k_f = scal_ref[6] * mxs + scal_ref[7] * mys + scal_ref[8] * mzs
    i_i = jnp.round(i_f).astype(jnp.int32)
    j_i = jnp.round(j_f).astype(jnp.int32)
    k_i = jnp.round(k_f).astype(jnp.int32)
    oob = ((i_i < 0) | (i_i >= n_x) | (j_i < 0) | (j_i >= n_y)
           | (k_i < 0) | (k_i >= n_z))
    flat = i_i * (n_y * n_z) + j_i * n_z + k_i
    idx_ref[...] = jnp.where(oob, 0, flat)
    w_ref[...] = jnp.where(oob, 0.0, seg_len)


def _geometry(t_sorted, ray_arrs, scal, n_x, n_y, n_z, block_rays,
              blk0, nb_piece):
    n_ray, n_int = t_sorted.shape

    ray_spec = pl.BlockSpec((1, block_rays), lambda i: (0, i + blk0))
    out_spec = pl.BlockSpec((n_int, block_rays), lambda i: (i, 0))
    return pl.pallas_call(
        functools.partial(_geom_body, n_x=n_x, n_y=n_y, n_z=n_z),
        grid=(nb_piece,),
        in_specs=[
            pl.BlockSpec(memory_space=pltpu.SMEM),
            pl.BlockSpec((block_rays, n_int), lambda i: (i + blk0, 0)),
            ray_spec, ray_spec, ray_spec, ray_spec, ray_spec, ray_spec,
        ],
        out_specs=[out_spec, out_spec],
        out_shape=[
            jax.ShapeDtypeStruct((nb_piece * n_int, block_rays), jnp.int32),
            jax.ShapeDtypeStruct((nb_piece * n_int, block_rays), jnp.float32),
        ],
    )(scal, t_sorted, *ray_arrs)


def _sc_gather(vol_flat, idx, w, n_ray, n_int, block_rays):
    blk_elems = n_int * block_rays
    nb = idx.shape[0] // blk_elems
    info = plsc.get_sparse_core_info()
    nc, ns, nl = info.num_cores, info.num_subcores, info.num_lanes
    nw = nc * ns
    chunks_per_worker = nb // nw
    ngroups = block_rays // nl

    mesh = plsc.VectorSubcoreMesh(core_axis_name="c", subcore_axis_name="s")

    @functools.partial(
        pl.kernel,
        out_type=jax.ShapeDtypeStruct((n_ray,), jnp.float32),
        mesh=mesh,
        scratch_types=[
            pltpu.VMEM((blk_elems,), jnp.int32),
            pltpu.VMEM((blk_elems,), jnp.float32),
            pltpu.VMEM((blk_elems,), jnp.float32),
            pltpu.VMEM((block_rays,), jnp.float32),
            pltpu.SemaphoreType.DMA,
        ],
    )
    def run(vol_hbm, idx_hbm, w_hbm, out_hbm, idx_v, w_v, vals_v, out_v, sem):
        wid = lax.axis_index("s") * nc + lax.axis_index("c")

        def chunk_body(c, _):
            blk = wid * chunks_per_worker + c
            el0 = blk * blk_elems
            pltpu.sync_copy(idx_hbm.at[pl.ds(el0, blk_elems)], idx_v)
            pltpu.sync_copy(w_hbm.at[pl.ds(el0, blk_elems)], w_v)
            pltpu.async_copy(vol_hbm.at[idx_v], vals_v, sem).wait()

            def seg_body(s, accs):
                base = s * block_rays
                return tuple(
                    accs[g] + vals_v[pl.ds(base + g * nl, nl)]
                    * w_v[pl.ds(base + g * nl, nl)]
                    for g in range(ngroups)
                )

            accs = tuple(jnp.zeros((nl,), jnp.float32) for _ in range(ngroups))
            accs = lax.fori_loop(0, n_int, seg_body, accs)
            for g in range(ngroups):
                out_v[g * nl:(g + 1) * nl] = accs[g]
            pltpu.sync_copy(out_v, out_hbm.at[pl.ds(blk * block_rays,
                                                    block_rays)])
            return 0

        lax.fori_loop(0, chunks_per_worker, chunk_body, 0)

    return run(vol_flat, idx, w)


def kernel(volume, t_sorted, M, b, src, dst):
    n_x, n_y, n_z = volume.shape
    n_ray, n_int = t_sorted.shape
    m_inv = jnp.linalg.inv(M)
    block_rays = 128
    n_pieces = 4

    sx = src[:, 0].reshape(1, n_ray); sy = src[:, 1].reshape(1, n_ray)
    sz = src[:, 2].reshape(1, n_ray)
    ex = dst[:, 0].reshape(1, n_ray); ey = dst[:, 1].reshape(1, n_ray)
    ez = dst[:, 2].reshape(1, n_ray)
    ray_arrs = (sx, sy, sz, ex, ey, ez)
    scal = jnp.concatenate([m_inv.reshape(-1), b]).astype(jnp.float32)
    vol_flat = volume.reshape(-1)

    nb = n_ray // block_rays
    nb_piece = nb // n_pieces
    rays_piece = n_ray // n_pieces
    outs = []
    for p in range(n_pieces):
        idx, w = _geometry(t_sorted, ray_arrs, scal, n_x, n_y, n_z,
                           block_rays, p * nb_piece, nb_piece)
        outs.append(_sc_gather(vol_flat, idx.reshape(-1), w.reshape(-1),
                               rays_piece, n_int, block_rays))
    return jnp.concatenate(outs)


# speedup vs baseline: 1.8697x; 1.0475x over previous
import functools

import jax
import jax.numpy as jnp
from jax import lax
from jax.experimental import pallas as pl
from jax.experimental.pallas import tpu as pltpu
from jax.experimental.pallas import tpu_sc as plsc


def _geom_body(scal_ref, t_ref, sx_ref, sy_ref, sz_ref, ex_ref, ey_ref,
               ez_ref, idx_ref, w_ref, *, n_x, n_y, n_z):
    n_int = t_ref.shape[1]
    t0 = t_ref[...].T
    t1 = jnp.concatenate([t0[1:n_int], t0[n_int - 1:n_int]], axis=0)

    sx = sx_ref[...]; sy = sy_ref[...]; sz = sz_ref[...]
    dx = ex_ref[...] - sx
    dy = ey_ref[...] - sy
    dz = ez_ref[...] - sz

    x0 = sx + t0 * dx; x1 = sx + t1 * dx
    y0 = sy + t0 * dy; y1 = sy + t1 * dy
    z0 = sz + t0 * dz; z1 = sz + t1 * dz

    seg_len = jnp.sqrt((x1 - x0) ** 2 + (y1 - y0) ** 2 + (z1 - z0) ** 2)

    mx = 0.5 * (x0 + x1)
    my = 0.5 * (y0 + y1)
    mz = 0.5 * (z0 + z1)
    mxs = mx - scal_ref[9]
    mys = my - scal_ref[10]
    mzs = mz - scal_ref[11]
    i_f = scal_ref[0] * mxs + scal_ref[1] * mys + scal_ref[2] * mzs
    j_f = scal_ref[3] * mxs + scal_ref[4] * mys + scal_ref[5] * mzs
    k_f = scal_ref[6] * mxs + scal_ref[7] * mys + scal_ref[8] * mzs
    i_i = jnp.round(i_f).astype(jnp.int32)
    j_i = jnp.round(j_f).astype(jnp.int32)
    k_i = jnp.round(k_f).astype(jnp.int32)
    oob = ((i_i < 0) | (i_i >= n_x) | (j_i < 0) | (j_i >= n_y)
           | (k_i < 0) | (k_i >= n_z))
    flat = i_i * (n_y * n_z) + j_i * n_z + k_i
    idx_ref[...] = jnp.where(oob, 0, flat)
    w_ref[...] = jnp.where(oob, 0.0, seg_len)


def _geometry(t_sorted, ray_arrs, scal, n_x, n_y, n_z, block_rays,
              blk0, nb_piece):
    n_ray, n_int = t_sorted.shape

    ray_spec = pl.BlockSpec((1, block_rays), lambda i: (0, i + blk0))
    out_spec = pl.BlockSpec((n_int, block_rays), lambda i: (i, 0))
    return pl.pallas_call(
        functools.partial(_geom_body, n_x=n_x, n_y=n_y, n_z=n_z),
        grid=(nb_piece,),
        in_specs=[
            pl.BlockSpec(memory_space=pltpu.SMEM),
            pl.BlockSpec((block_rays, n_int), lambda i: (i + blk0, 0)),
            ray_spec, ray_spec, ray_spec, ray_spec, ray_spec, ray_spec,
        ],
        out_specs=[out_spec, out_spec],
        out_shape=[
            jax.ShapeDtypeStruct((nb_piece * n_int, block_rays), jnp.int32),
            jax.ShapeDtypeStruct((nb_piece * n_int, block_rays), jnp.float32),
        ],
    )(scal, t_sorted, *ray_arrs)


def _sc_gather(vol_flat, idx, w, n_ray, n_int, block_rays):
    blk_elems = n_int * block_rays
    nb = idx.shape[0] // blk_elems
    info = plsc.get_sparse_core_info()
    nc, ns, nl = info.num_cores, info.num_subcores, info.num_lanes
    nw = nc * ns
    chunks_per_worker = nb // nw
    ngroups = block_rays // nl

    mesh = plsc.VectorSubcoreMesh(core_axis_name="c", subcore_axis_name="s")

    @functools.partial(
        pl.kernel,
        out_type=jax.ShapeDtypeStruct((n_ray,), jnp.float32),
        mesh=mesh,
        scratch_types=[
            pltpu.VMEM((blk_elems,), jnp.int32),
            pltpu.VMEM((blk_elems,), jnp.float32),
            pltpu.VMEM((blk_elems,), jnp.float32),
            pltpu.VMEM((block_rays,), jnp.float32),
            pltpu.SemaphoreType.DMA,
        ],
    )
    def run(vol_hbm, idx_hbm, w_hbm, out_hbm, idx_v, w_v, vals_v, out_v, sem):
        wid = lax.axis_index("s") * nc + lax.axis_index("c")

        def chunk_body(c, _):
            blk = wid * chunks_per_worker + c
            el0 = blk * blk_elems
            pltpu.sync_copy(idx_hbm.at[pl.ds(el0, blk_elems)], idx_v)
            pltpu.sync_copy(w_hbm.at[pl.ds(el0, blk_elems)], w_v)
            pltpu.async_copy(vol_hbm.at[idx_v], vals_v, sem).wait()

            def seg_body(s, accs):
                base = s * block_rays
                return tuple(
                    accs[g] + vals_v[pl.ds(base + g * nl, nl)]
                    * w_v[pl.ds(base + g * nl, nl)]
                    for g in range(ngroups)
                )

            accs = tuple(jnp.zeros((nl,), jnp.float32) for _ in range(ngroups))
            accs = lax.fori_loop(0, n_int, seg_body, accs)
            for g in range(ngroups):
                out_v[g * nl:(g + 1) * nl] = accs[g]
            pltpu.sync_copy(out_v, out_hbm.at[pl.ds(blk * block_rays,
                                                    block_rays)])
            return 0

        lax.fori_loop(0, chunks_per_worker, chunk_body, 0)

    return run(vol_flat, idx, w)


def kernel(volume, t_sorted, M, b, src, dst):
    n_x, n_y, n_z = volume.shape
    n_ray, n_int = t_sorted.shape
    m_inv = jnp.linalg.inv(M)
    block_rays = 128
    n_pieces = 8

    sx = src[:, 0].reshape(1, n_ray); sy = src[:, 1].reshape(1, n_ray)
    sz = src[:, 2].reshape(1, n_ray)
    ex = dst[:, 0].reshape(1, n_ray); ey = dst[:, 1].reshape(1, n_ray)
    ez = dst[:, 2].reshape(1, n_ray)
    ray_arrs = (sx, sy, sz, ex, ey, ez)
    scal = jnp.concatenate([m_inv.reshape(-1), b]).astype(jnp.float32)
    vol_flat = volume.reshape(-1)

    nb = n_ray // block_rays
    nb_piece = nb // n_pieces
    rays_piece = n_ray // n_pieces
    outs = []
    for p in range(n_pieces):
        idx, w = _geometry(t_sorted, ray_arrs, scal, n_x, n_y, n_z,
                           block_rays, p * nb_piece, nb_piece)
        outs.append(_sc_gather(vol_flat, idx.reshape(-1), w.reshape(-1),
                               rays_piece, n_int, block_rays))
    return jnp.concatenate(outs)


# speedup vs baseline: 1.9528x; 1.0444x over previous
import functools

import jax
import jax.numpy as jnp
from jax import lax
from jax.experimental import pallas as pl
from jax.experimental.pallas import tpu as pltpu
from jax.experimental.pallas import tpu_sc as plsc


def _geom_body(scal_ref, t_ref, sx_ref, sy_ref, sz_ref, ex_ref, ey_ref,
               ez_ref, idx_ref, w_ref, *, n_x, n_y, n_z):
    n_int = t_ref.shape[1]
    t0 = t_ref[...].T
    t1 = jnp.concatenate([t0[1:n_int], t0[n_int - 1:n_int]], axis=0)

    sx = sx_ref[...]; sy = sy_ref[...]; sz = sz_ref[...]
    dx = ex_ref[...] - sx
    dy = ey_ref[...] - sy
    dz = ez_ref[...] - sz

    chord = jnp.sqrt(dx * dx + dy * dy + dz * dz)
    seg_len = (t1 - t0) * chord

    x0 = sx + t0 * dx; x1 = sx + t1 * dx
    y0 = sy + t0 * dy; y1 = sy + t1 * dy
    z0 = sz + t0 * dz; z1 = sz + t1 * dz
    mxs = 0.5 * (x0 + x1) - scal_ref[9]
    mys = 0.5 * (y0 + y1) - scal_ref[10]
    mzs = 0.5 * (z0 + z1) - scal_ref[11]
    i_i = jnp.round(scal_ref[0] * mxs).astype(jnp.int32)
    j_i = jnp.round(scal_ref[4] * mys).astype(jnp.int32)
    k_i = jnp.round(scal_ref[8] * mzs).astype(jnp.int32)
    oob = (i_i >= n_x) | (j_i >= n_y) | (k_i >= n_z)
    flat = i_i * (n_y * n_z) + j_i * n_z + k_i
    idx_ref[...] = jnp.where(oob, 0, flat)
    w_ref[...] = jnp.where(oob, 0.0, seg_len)


def _geometry(t_sorted, ray_arrs, scal, n_x, n_y, n_z, block_rays,
              blk0, nb_piece):
    n_ray, n_int = t_sorted.shape

    ray_spec = pl.BlockSpec((1, block_rays), lambda i: (0, i + blk0))
    out_spec = pl.BlockSpec((n_int, block_rays), lambda i: (i, 0))
    return pl.pallas_call(
        functools.partial(_geom_body, n_x=n_x, n_y=n_y, n_z=n_z),
        grid=(nb_piece,),
        in_specs=[
            pl.BlockSpec(memory_space=pltpu.SMEM),
            pl.BlockSpec((block_rays, n_int), lambda i: (i + blk0, 0)),
            ray_spec, ray_spec, ray_spec, ray_spec, ray_spec, ray_spec,
        ],
        out_specs=[out_spec, out_spec],
        out_shape=[
            jax.ShapeDtypeStruct((nb_piece * n_int, block_rays), jnp.int32),
            jax.ShapeDtypeStruct((nb_piece * n_int, block_rays), jnp.float32),
        ],
    )(scal, t_sorted, *ray_arrs)


def _sc_gather(vol_flat, idx, w, n_ray, n_int, block_rays):
    blk_elems = n_int * block_rays
    nb = idx.shape[0] // blk_elems
    info = plsc.get_sparse_core_info()
    nc, ns, nl = info.num_cores, info.num_subcores, info.num_lanes
    nw = nc * ns
    cpw = nb // nw
    ngroups = block_rays // nl

    mesh = plsc.VectorSubcoreMesh(core_axis_name="c", subcore_axis_name="s")

    @functools.partial(
        pl.kernel,
        out_type=jax.ShapeDtypeStruct((n_ray,), jnp.float32),
        mesh=mesh,
        scratch_types=[
            pltpu.VMEM((blk_elems,), jnp.int32),
            pltpu.VMEM((blk_elems,), jnp.int32),
            pltpu.VMEM((blk_elems,), jnp.float32),
            pltpu.VMEM((blk_elems,), jnp.float32),
            pltpu.VMEM((blk_elems,), jnp.float32),
            pltpu.VMEM((blk_elems,), jnp.float32),
            pltpu.VMEM((block_rays,), jnp.float32),
            pltpu.SemaphoreType.DMA((2,)),
            pltpu.SemaphoreType.DMA((2,)),
        ],
    )
    def run(vol_hbm, idx_hbm, w_hbm, out_hbm, idx_v0, idx_v1, w_v0, w_v1,
            vals_v0, vals_v1, out_v, sem_in, sem_g):
        wid = lax.axis_index("s") * nc + lax.axis_index("c")
        idx_vs = (idx_v0, idx_v1)
        w_vs = (w_v0, w_v1)
        vals_vs = (vals_v0, vals_v1)

        def fire_inputs(c, slot):
            el0 = (wid * cpw + c) * blk_elems
            hi = pltpu.async_copy(idx_hbm.at[pl.ds(el0, blk_elems)],
                                  idx_vs[slot], sem_in.at[slot])
            hw = pltpu.async_copy(w_hbm.at[pl.ds(el0, blk_elems)],
                                  w_vs[slot], sem_in.at[slot])
            return hi, hw

        def fire_gather(slot):
            return pltpu.async_copy(vol_hbm.at[idx_vs[slot]],
                                    vals_vs[slot], sem_g.at[slot])

        pending = {}
        for c in range(min(2, cpw)):
            pending[c] = fire_inputs(c, c % 2)

        hi, hw = pending.pop(0)
        hi.wait(); hw.wait()
        gathers = {0: fire_gather(0)}

        for c in range(cpw):
            slot = c % 2
            if c + 1 < cpw:
                hi, hw = pending.pop(c + 1)
                hi.wait(); hw.wait()
                gathers[c + 1] = fire_gather((c + 1) % 2)
            gathers.pop(c).wait()

            vv = vals_vs[slot]
            wv = w_vs[slot]

            def seg_body(s, accs):
                base = s * block_rays
                return tuple(
                    accs[g] + vv[pl.ds(base + g * nl, nl)]
                    * wv[pl.ds(base + g * nl, nl)]
                    for g in range(ngroups)
                )

            accs = tuple(jnp.zeros((nl,), jnp.float32) for _ in range(ngroups))
            accs = lax.fori_loop(0, n_int, seg_body, accs)
            for g in range(ngroups):
                out_v[g * nl:(g + 1) * nl] = accs[g]
            pltpu.sync_copy(out_v,
                            out_hbm.at[pl.ds((wid * cpw + c) * block_rays,
                                             block_rays)])
            if c + 2 < cpw:
                pending[c + 2] = fire_inputs(c + 2, slot)

    return run(vol_flat, idx, w)


def kernel(volume, t_sorted, M, b, src, dst):
    n_x, n_y, n_z = volume.shape
    n_ray, n_int = t_sorted.shape
    m_inv = jnp.linalg.inv(M)
    block_rays = 128
    n_pieces = 8

    sx = src[:, 0].reshape(1, n_ray); sy = src[:, 1].reshape(1, n_ray)
    sz = src[:, 2].reshape(1, n_ray)
    ex = dst[:, 0].reshape(1, n_ray); ey = dst[:, 1].reshape(1, n_ray)
    ez = dst[:, 2].reshape(1, n_ray)
    ray_arrs = (sx, sy, sz, ex, ey, ez)
    scal = jnp.concatenate([m_inv.reshape(-1), b]).astype(jnp.float32)
    vol_flat = volume.reshape(-1)

    nb = n_ray // block_rays
    nb_piece = nb // n_pieces
    rays_piece = n_ray // n_pieces
    outs = []
    for p in range(n_pieces):
        idx, w = _geometry(t_sorted, ray_arrs, scal, n_x, n_y, n_z,
                           block_rays, p * nb_piece, nb_piece)
        outs.append(_sc_gather(vol_flat, idx.reshape(-1), w.reshape(-1),
                               rays_piece, n_int, block_rays))
    return jnp.concatenate(outs)


# speedup vs baseline: 1.9758x; 1.0118x over previous
import functools

import jax
import jax.numpy as jnp
from jax import lax
from jax.experimental import pallas as pl
from jax.experimental.pallas import tpu as pltpu
from jax.experimental.pallas import tpu_sc as plsc


def _geom_body(scal_ref, t_ref, sx_ref, sy_ref, sz_ref, ex_ref, ey_ref,
               ez_ref, idx_ref, w_ref, *, n_x, n_y, n_z):
    n_int = t_ref.shape[1]
    t0 = t_ref[...].T
    t1 = jnp.concatenate([t0[1:n_int], t0[n_int - 1:n_int]], axis=0)

    sx = sx_ref[...]; sy = sy_ref[...]; sz = sz_ref[...]
    dx = ex_ref[...] - sx
    dy = ey_ref[...] - sy
    dz = ez_ref[...] - sz

    chord = jnp.sqrt(dx * dx + dy * dy + dz * dz)
    seg_len = (t1 - t0) * chord

    x0 = sx + t0 * dx; x1 = sx + t1 * dx
    y0 = sy + t0 * dy; y1 = sy + t1 * dy
    z0 = sz + t0 * dz; z1 = sz + t1 * dz
    mxs = 0.5 * (x0 + x1) - scal_ref[9]
    mys = 0.5 * (y0 + y1) - scal_ref[10]
    mzs = 0.5 * (z0 + z1) - scal_ref[11]
    i_i = jnp.round(scal_ref[0] * mxs).astype(jnp.int32)
    j_i = jnp.round(scal_ref[4] * mys).astype(jnp.int32)
    k_i = jnp.round(scal_ref[8] * mzs).astype(jnp.int32)
    oob = (i_i >= n_x) | (j_i >= n_y) | (k_i >= n_z)
    flat = i_i * (n_y * n_z) + j_i * n_z + k_i
    idx = jnp.where(oob, 0, flat)
    w = jnp.where(oob, 0.0, seg_len)
    bw = idx_ref.shape[1]
    for bb in range(idx.shape[1] // bw):
        idx_ref[bb * n_int:(bb + 1) * n_int] = idx[:, bb * bw:(bb + 1) * bw]
        w_ref[bb * n_int:(bb + 1) * n_int] = w[:, bb * bw:(bb + 1) * bw]


def _geometry(t_sorted, ray_arrs, scal, n_x, n_y, n_z, block_rays,
              blk0, nb_piece, rpg):
    n_ray, n_int = t_sorted.shape
    g0 = blk0 // rpg
    rays_step = rpg * block_rays

    ray_spec = pl.BlockSpec((1, rays_step), lambda i: (0, i + g0))
    out_spec = pl.BlockSpec((rpg * n_int, block_rays), lambda i: (i, 0))
    return pl.pallas_call(
        functools.partial(_geom_body, n_x=n_x, n_y=n_y, n_z=n_z),
        grid=(nb_piece // rpg,),
        in_specs=[
            pl.BlockSpec(memory_space=pltpu.SMEM),
            pl.BlockSpec((rays_step, n_int), lambda i: (i + g0, 0)),
            ray_spec, ray_spec, ray_spec, ray_spec, ray_spec, ray_spec,
        ],
        out_specs=[out_spec, out_spec],
        out_shape=[
            jax.ShapeDtypeStruct((nb_piece * n_int, block_rays), jnp.int32),
            jax.ShapeDtypeStruct((nb_piece * n_int, block_rays), jnp.float32),
        ],
    )(scal, t_sorted, *ray_arrs)


def _sc_gather(vol_flat, idx, w, n_ray, n_int, block_rays):
    blk_elems = n_int * block_rays
    nb = idx.shape[0] // blk_elems
    info = plsc.get_sparse_core_info()
    nc, ns, nl = info.num_cores, info.num_subcores, info.num_lanes
    nw = nc * ns
    cpw = nb // nw
    ngroups = block_rays // nl

    mesh = plsc.VectorSubcoreMesh(core_axis_name="c", subcore_axis_name="s")

    @functools.partial(
        pl.kernel,
        out_type=jax.ShapeDtypeStruct((n_ray,), jnp.float32),
        mesh=mesh,
        scratch_types=[
            pltpu.VMEM((blk_elems,), jnp.int32),
            pltpu.VMEM((blk_elems,), jnp.int32),
            pltpu.VMEM((blk_elems,), jnp.float32),
            pltpu.VMEM((blk_elems,), jnp.float32),
            pltpu.VMEM((blk_elems,), jnp.float32),
            pltpu.VMEM((blk_elems,), jnp.float32),
            pltpu.VMEM((block_rays,), jnp.float32),
            pltpu.SemaphoreType.DMA((2,)),
            pltpu.SemaphoreType.DMA((2,)),
        ],
    )
    def run(vol_hbm, idx_hbm, w_hbm, out_hbm, idx_v0, idx_v1, w_v0, w_v1,
            vals_v0, vals_v1, out_v, sem_in, sem_g):
        wid = lax.axis_index("s") * nc + lax.axis_index("c")
        idx_vs = (idx_v0, idx_v1)
        w_vs = (w_v0, w_v1)
        vals_vs = (vals_v0, vals_v1)

        def fire_inputs(c, slot):
            el0 = (wid * cpw + c) * blk_elems
            hi = pltpu.async_copy(idx_hbm.at[pl.ds(el0, blk_elems)],
                                  idx_vs[slot], sem_in.at[slot])
            hw = pltpu.async_copy(w_hbm.at[pl.ds(el0, blk_elems)],
                                  w_vs[slot], sem_in.at[slot])
            return hi, hw

        def fire_gather(slot):
            return pltpu.async_copy(vol_hbm.at[idx_vs[slot]],
                                    vals_vs[slot], sem_g.at[slot])

        pending = {}
        for c in range(min(2, cpw)):
            pending[c] = fire_inputs(c, c % 2)

        hi, hw = pending.pop(0)
        hi.wait(); hw.wait()
        gathers = {0: fire_gather(0)}

        for c in range(cpw):
            slot = c % 2
            if c + 1 < cpw:
                hi, hw = pending.pop(c + 1)
                hi.wait(); hw.wait()
                gathers[c + 1] = fire_gather((c + 1) % 2)
            gathers.pop(c).wait()

            vv = vals_vs[slot]
            wv = w_vs[slot]

            def seg_body(s, accs):
                base = s * block_rays
                return tuple(
                    accs[g] + vv[pl.ds(base + g * nl, nl)]
                    * wv[pl.ds(base + g * nl, nl)]
                    for g in range(ngroups)
                )

            accs = tuple(jnp.zeros((nl,), jnp.float32) for _ in range(ngroups))
            accs = lax.fori_loop(0, n_int, seg_body, accs)
            for g in range(ngroups):
                out_v[g * nl:(g + 1) * nl] = accs[g]
            pltpu.sync_copy(out_v,
                            out_hbm.at[pl.ds((wid * cpw + c) * block_rays,
                                             block_rays)])
            if c + 2 < cpw:
                pending[c + 2] = fire_inputs(c + 2, slot)

    return run(vol_flat, idx, w)


def kernel(volume, t_sorted, M, b, src, dst):
    n_x, n_y, n_z = volume.shape
    n_ray, n_int = t_sorted.shape
    m_inv = jnp.linalg.inv(M)
    block_rays = 128
    n_pieces = 8

    sx = src[:, 0].reshape(1, n_ray); sy = src[:, 1].reshape(1, n_ray)
    sz = src[:, 2].reshape(1, n_ray)
    ex = dst[:, 0].reshape(1, n_ray); ey = dst[:, 1].reshape(1, n_ray)
    ez = dst[:, 2].reshape(1, n_ray)
    ray_arrs = (sx, sy, sz, ex, ey, ez)
    scal = jnp.concatenate([m_inv.reshape(-1), b]).astype(jnp.float32)
    vol_flat = volume.reshape(-1)

    nb = n_ray // block_rays
    nb_piece = nb // n_pieces
    rays_piece = n_ray // n_pieces
    outs = []
    for p in range(n_pieces):
        idx, w = _geometry(t_sorted, ray_arrs, scal, n_x, n_y, n_z,
                           block_rays, p * nb_piece, nb_piece, 4)
        outs.append(_sc_gather(vol_flat, idx.reshape(-1), w.reshape(-1),
                               rays_piece, n_int, block_rays))
    return jnp.concatenate(outs)


# speedup vs baseline: 2.0698x; 1.0476x over previous
import functools

import jax
import jax.numpy as jnp
from jax import lax
from jax.experimental import pallas as pl
from jax.experimental.pallas import tpu as pltpu
from jax.experimental.pallas import tpu_sc as plsc


def _geom_body(scal_ref, t_ref, sx_ref, sy_ref, sz_ref, ex_ref, ey_ref,
               ez_ref, idx_ref, w_ref, *, n_x, n_y, n_z):
    n_int = t_ref.shape[1]
    t0 = t_ref[...].T
    t1 = jnp.concatenate([t0[1:n_int], t0[n_int - 1:n_int]], axis=0)

    sx = sx_ref[...]; sy = sy_ref[...]; sz = sz_ref[...]
    dx = ex_ref[...] - sx
    dy = ey_ref[...] - sy
    dz = ez_ref[...] - sz

    chord = jnp.sqrt(dx * dx + dy * dy + dz * dz)
    seg_len = (t1 - t0) * chord

    x0 = sx + t0 * dx; x1 = sx + t1 * dx
    y0 = sy + t0 * dy; y1 = sy + t1 * dy
    z0 = sz + t0 * dz; z1 = sz + t1 * dz
    mxs = 0.5 * (x0 + x1) - scal_ref[9]
    mys = 0.5 * (y0 + y1) - scal_ref[10]
    mzs = 0.5 * (z0 + z1) - scal_ref[11]
    i_i = jnp.round(scal_ref[0] * mxs).astype(jnp.int32)
    j_i = jnp.round(scal_ref[4] * mys).astype(jnp.int32)
    k_i = jnp.round(scal_ref[8] * mzs).astype(jnp.int32)
    oob = (i_i >= n_x) | (j_i >= n_y) | (k_i >= n_z)
    flat = i_i * (n_y * n_z) + j_i * n_z + k_i
    idx = jnp.where(oob, 0, flat)
    w = jnp.where(oob, 0.0, seg_len)
    bw = idx_ref.shape[1]
    for bb in range(idx.shape[1] // bw):
        idx_ref[bb * n_int:(bb + 1) * n_int] = idx[:, bb * bw:(bb + 1) * bw]
        w_ref[bb * n_int:(bb + 1) * n_int] = w[:, bb * bw:(bb + 1) * bw]


def _geometry(t_sorted, ray_arrs, scal, n_x, n_y, n_z, block_rays,
              blk0, nb_piece, rpg):
    n_ray, n_int = t_sorted.shape
    g0 = blk0 // rpg
    rays_step = rpg * block_rays

    ray_spec = pl.BlockSpec((1, rays_step), lambda i: (0, i + g0))
    out_spec = pl.BlockSpec((rpg * n_int, block_rays), lambda i: (i, 0))
    return pl.pallas_call(
        functools.partial(_geom_body, n_x=n_x, n_y=n_y, n_z=n_z),
        grid=(nb_piece // rpg,),
        in_specs=[
            pl.BlockSpec(memory_space=pltpu.SMEM),
            pl.BlockSpec((rays_step, n_int), lambda i: (i + g0, 0)),
            ray_spec, ray_spec, ray_spec, ray_spec, ray_spec, ray_spec,
        ],
        out_specs=[out_spec, out_spec],
        out_shape=[
            jax.ShapeDtypeStruct((nb_piece * n_int, block_rays), jnp.int32),
            jax.ShapeDtypeStruct((nb_piece * n_int, block_rays), jnp.float32),
        ],
    )(scal, t_sorted, *ray_arrs)


def _sc_gather(vol_flat, idx, w, n_ray, n_int, block_rays):
    blk_elems = n_int * block_rays
    nb = idx.shape[0] // blk_elems
    info = plsc.get_sparse_core_info()
    nc, ns, nl = info.num_cores, info.num_subcores, info.num_lanes
    nw = nc * ns
    cpw = nb // nw
    ngroups = block_rays // nl

    mesh = plsc.VectorSubcoreMesh(core_axis_name="c", subcore_axis_name="s")

    @functools.partial(
        pl.kernel,
        out_type=jax.ShapeDtypeStruct((n_ray,), jnp.float32),
        mesh=mesh,
        scratch_types=[
            pltpu.VMEM((blk_elems,), jnp.int32),
            pltpu.VMEM((blk_elems,), jnp.int32),
            pltpu.VMEM((blk_elems,), jnp.float32),
            pltpu.VMEM((blk_elems,), jnp.float32),
            pltpu.VMEM((blk_elems,), jnp.float32),
            pltpu.VMEM((blk_elems,), jnp.float32),
            pltpu.VMEM((block_rays,), jnp.float32),
            pltpu.SemaphoreType.DMA((2,)),
            pltpu.SemaphoreType.DMA((2,)),
        ],
    )
    def run(vol_hbm, idx_hbm, w_hbm, out_hbm, idx_v0, idx_v1, w_v0, w_v1,
            vals_v0, vals_v1, out_v, sem_in, sem_g):
        wid = lax.axis_index("s") * nc + lax.axis_index("c")
        idx_vs = (idx_v0, idx_v1)
        w_vs = (w_v0, w_v1)
        vals_vs = (vals_v0, vals_v1)

        def fire_inputs(c, slot):
            el0 = (wid * cpw + c) * blk_elems
            hi = pltpu.async_copy(idx_hbm.at[pl.ds(el0, blk_elems)],
                                  idx_vs[slot], sem_in.at[slot])
            hw = pltpu.async_copy(w_hbm.at[pl.ds(el0, blk_elems)],
                                  w_vs[slot], sem_in.at[slot])
            return hi, hw

        def fire_gather(slot):
            return pltpu.async_copy(vol_hbm.at[idx_vs[slot]],
                                    vals_vs[slot], sem_g.at[slot])

        pending = {}
        for c in range(min(2, cpw)):
            pending[c] = fire_inputs(c, c % 2)

        hi, hw = pending.pop(0)
        hi.wait(); hw.wait()
        gathers = {0: fire_gather(0)}

        for c in range(cpw):
            slot = c % 2
            if c + 1 < cpw:
                hi, hw = pending.pop(c + 1)
                hi.wait(); hw.wait()
                gathers[c + 1] = fire_gather((c + 1) % 2)
            gathers.pop(c).wait()

            vv = vals_vs[slot]
            wv = w_vs[slot]

            def seg_body(s, accs):
                base = s * block_rays
                return tuple(
                    accs[g] + vv[pl.ds(base + g * nl, nl)]
                    * wv[pl.ds(base + g * nl, nl)]
                    for g in range(ngroups)
                )

            accs = tuple(jnp.zeros((nl,), jnp.float32) for _ in range(ngroups))
            accs = lax.fori_loop(0, n_int, seg_body, accs)
            for g in range(ngroups):
                out_v[g * nl:(g + 1) * nl] = accs[g]
            pltpu.sync_copy(out_v,
                            out_hbm.at[pl.ds((wid * cpw + c) * block_rays,
                                             block_rays)])
            if c + 2 < cpw:
                pending[c + 2] = fire_inputs(c + 2, slot)

    return run(vol_flat, idx, w)


def kernel(volume, t_sorted, M, b, src, dst):
    n_x, n_y, n_z = volume.shape
    n_ray, n_int = t_sorted.shape
    m_inv = jnp.linalg.inv(M)
    block_rays = 128
    n_pieces = 4

    sx = src[:, 0].reshape(1, n_ray); sy = src[:, 1].reshape(1, n_ray)
    sz = src[:, 2].reshape(1, n_ray)
    ex = dst[:, 0].reshape(1, n_ray); ey = dst[:, 1].reshape(1, n_ray)
    ez = dst[:, 2].reshape(1, n_ray)
    ray_arrs = (sx, sy, sz, ex, ey, ez)
    scal = jnp.concatenate([m_inv.reshape(-1), b]).astype(jnp.float32)
    vol_flat = volume.reshape(-1)

    nb = n_ray // block_rays
    nb_piece = nb // n_pieces
    rays_piece = n_ray // n_pieces
    outs = []
    for p in range(n_pieces):
        idx, w = _geometry(t_sorted, ray_arrs, scal, n_x, n_y, n_z,
                           block_rays, p * nb_piece, nb_piece, 4)
        outs.append(_sc_gather(vol_flat, idx.reshape(-1), w.reshape(-1),
                               rays_piece, n_int, block_rays))
    return jnp.concatenate(outs)
